# Initial kernel scaffold; baseline (speedup 1.0000x reference)
import jax
import jax.numpy as jnp
from jax.experimental import pallas as pl


def kernel(x, W1, b1, W2, b2):
    raise NotImplementedError("write your pallas kernel here")


# speedup vs baseline: 3.2795x; 3.2795x over previous
import functools

import jax
import jax.numpy as jnp
from jax import lax
from jax.experimental import pallas as pl
from jax.experimental.pallas import tpu as pltpu
from jax.experimental.pallas import tpu_sc as plsc

_B, _N, _D, _R, _C = 64, 8192, 128, 5, 5
_NC = 2048
_L = 16
_NW = 32
_RPW = _B // _NW


def _score_body(x_ref, w_ref, o_ref):
    xb = x_ref[0]
    w = w_ref[...]
    s = lax.dot_general(w, xb, (((1,), (1,)), ((), ())),
                        preferred_element_type=jnp.float32)
    o_ref[0, :, :] = s


def _scores(x, w1row):
    return pl.pallas_call(
        _score_body,
        grid=(_B,),
        in_specs=[
            pl.BlockSpec((1, _N, _D), lambda b: (b, 0, 0)),
            pl.BlockSpec((1, _D), lambda b: (0, 0)),
        ],
        out_specs=pl.BlockSpec((1, 1, _N), lambda b: (b, 0, 0)),
        out_shape=jax.ShapeDtypeStruct((_B, 1, _N), jnp.float32),
    )(x, w1row)


def _sc_topk_body(s_hbm, w2_hbm, out_hbm, s_v, w2_v, out_v):
    cid = lax.axis_index("c")
    sid = lax.axis_index("s")
    wid = sid * 2 + cid
    pltpu.sync_copy(s_hbm.at[pl.ds(wid * (_RPW * _N), _RPW * _N)], s_v)
    pltpu.sync_copy(w2_hbm, w2_v)

    neg = jnp.full((_L,), -jnp.inf, jnp.float32)
    pos = jnp.full((_L,), jnp.inf, jnp.float32)
    zeros = jnp.zeros((_L,), jnp.float32)
    lane = lax.iota(jnp.int32, _L)

    for rl in range(_RPW):
        def step(i, carry):
            t0, t1, t2, t3, t4, u0, u1, u2, u3, u4 = carry
            v = s_v[pl.ds(rl * _N + i * _L, _L)]
            a = v
            m = jnp.maximum(t0, a); a = jnp.minimum(t0, a); t0 = m
            m = jnp.maximum(t1, a); a = jnp.minimum(t1, a); t1 = m
            m = jnp.maximum(t2, a); a = jnp.minimum(t2, a); t2 = m
            m = jnp.maximum(t3, a); a = jnp.minimum(t3, a); t3 = m
            t4 = jnp.maximum(t4, a)
            b = v
            m = jnp.minimum(u0, b); b = jnp.maximum(u0, b); u0 = m
            m = jnp.minimum(u1, b); b = jnp.maximum(u1, b); u1 = m
            m = jnp.minimum(u2, b); b = jnp.maximum(u2, b); u2 = m
            m = jnp.minimum(u3, b); b = jnp.maximum(u3, b); u3 = m
            u4 = jnp.minimum(u4, b)
            return (t0, t1, t2, t3, t4, u0, u1, u2, u3, u4)

        t0, t1, t2, t3, t4, u0, u1, u2, u3, u4 = lax.fori_loop(
            0, _N // _L, step,
            (neg, neg, neg, neg, neg, pos, pos, pos, pos, pos),
            unroll=4,
        )

        fv = jnp.where(lane == 2 * _R, jnp.float32(1.0), zeros)
        for k in range(_R):
            m = jnp.max(t0)
            fv = jnp.where(lane == k, m, fv)
            sel = lane == plsc.all_reduce_ffs(t0 == m)
            t0 = jnp.where(sel, t1, t0)
            t1 = jnp.where(sel, t2, t1)
            t2 = jnp.where(sel, t3, t2)
            t3 = jnp.where(sel, t4, t3)
            t4 = jnp.where(sel, neg, t4)
        for k in range(_R):
            m = jnp.min(u0)
            fv = jnp.where(lane == 2 * _R - 1 - k, m, fv)
            sel = lane == plsc.all_reduce_ffs(u0 == m)
            u0 = jnp.where(sel, u1, u0)
            u1 = jnp.where(sel, u2, u1)
            u2 = jnp.where(sel, u3, u2)
            u3 = jnp.where(sel, u4, u3)
            u4 = jnp.where(sel, pos, u4)

        ov = zeros
        for c in range(_C):
            w = w2_v[pl.ds(c * _L, _L)]
            ov = jnp.where(lane == c, jnp.sum(fv * w), ov)
        out_v[pl.ds(rl * _L, _L)] = ov

    pltpu.sync_copy(out_v, out_hbm.at[pl.ds(wid * (_RPW * _L), _RPW * _L)])


@functools.cache
def _sc_topk():
    return functools.partial(
        pl.kernel,
        mesh=plsc.VectorSubcoreMesh(core_axis_name="c", subcore_axis_name="s"),
        compiler_params=pltpu.CompilerParams(needs_layout_passes=False),
        out_type=jax.ShapeDtypeStruct((_B * _L,), jnp.float32),
        scratch_types=[
            pltpu.VMEM((_RPW * _N,), jnp.float32),
            pltpu.VMEM((_C * _L,), jnp.float32),
            pltpu.VMEM((_RPW * _L,), jnp.float32),
        ],
    )(_sc_topk_body)


def kernel(x, W1, b1, W2, b2):
    s = _scores(x, W1)
    w2p = jnp.zeros((_C, _L), jnp.float32)
    w2p = w2p.at[:, : 2 * _R].set(W2)
    w2p = w2p.at[:, 2 * _R].set(b2 + b1[0] * jnp.sum(W2, axis=1))
    out16 = _sc_topk()(s.reshape(-1), w2p.reshape(-1))
    return out16.reshape(_B, _L)[:, :_C]
